# baseline (speedup 1.0000x reference)
import functools

import jax
import jax.numpy as jnp
from jax import lax
from jax.experimental import pallas as pl
from jax.experimental.pallas import tpu as pltpu
from jax.experimental.pallas import tpu_sc as plsc

NC = 2
NS = 16
NW = NC * NS
L = 16

B = 4096
S = 200
D = 32
V = 1000000
CB = 512
GRID = (V + CB - 1) // CB
VP = GRID * CB
BPW = B // NW
G = 4
GS = G * S
NG = BPW // G
STREAMS = ((0, 128), (128, 128), (256, 128), (384, 128),
           (512, 128), (640, 128), (768, 32))


def _sc_body(tok_hbm, table_hbm, out_hbm, tokf_v, rows0, rows1, out_v,
             sem0, sem1):
    cid = lax.axis_index("c")
    sid = lax.axis_index("s")
    wid = sid * NC + cid

    pltpu.sync_copy(tok_hbm.at[wid], tokf_v)

    rows = (rows0, rows1)
    sems = (sem0, sem1)

    def start_gather(g, b):
        tb = g * GS
        for off, ln in STREAMS:
            pltpu.async_copy(table_hbm.at[tokf_v.at[pl.ds(tb + off, ln)]],
                             rows[b].at[pl.ds(off, ln)], sems[b])

    def wait_gather(b):
        pltpu.make_async_copy(table_hbm.at[pl.ds(0, GS)], rows[b], sems[b]).wait()

    zero = jnp.zeros((L,), jnp.float32)
    lane = lax.iota(jnp.int32, 16)
    one = jnp.ones((L,), jnp.float32)

    def compute(g, b):
        rb = rows[b]

        def body2(r, accs):
            accs = list(accs)
            rr = r * 2
            for u in range(G):
                ub = u * S
                for k in range(2):
                    a = u * 4 + k * 2
                    accs[a] = accs[a] + rb[ub + rr + k, pl.ds(0, L)]
                    accs[a + 1] = accs[a + 1] + rb[ub + rr + k, pl.ds(L, L)]
            return tuple(accs)

        accs = lax.fori_loop(0, S // 2, body2, (zero,) * (4 * G))

        for u in range(G):
            s = g * G + u
            o0 = accs[u * 4] + accs[u * 4 + 2]
            o1 = accs[u * 4 + 1] + accs[u * 4 + 3]

            base = s * S
            cnt = zero
            for k in range(S // L):
                t = tokf_v[pl.ds(base + k * L, L)]
                cnt = cnt + jnp.where(t != 0, 1.0, 0.0)
            t = tokf_v[pl.ds(base + S - L, L)]
            cnt = cnt + jnp.where((t != 0) & (lane >= L - S % L), 1.0, 0.0)

            inv = 1.0 / jnp.maximum(jnp.broadcast_to(jnp.sum(cnt), (L,)), one)
            out_v[s, pl.ds(0, L)] = o0 * inv
            out_v[s, pl.ds(L, L)] = o1 * inv

    start_gather(0, 0)
    start_gather(1, 1)

    def pair(i, _):
        g0 = i * 2
        wait_gather(0)
        compute(g0, 0)

        @pl.when(g0 + 2 < NG)
        def _():
            start_gather(g0 + 2, 0)

        wait_gather(1)
        compute(g0 + 1, 1)

        @pl.when(g0 + 3 < NG)
        def _():
            start_gather(g0 + 3, 1)

        return 0

    lax.fori_loop(0, NG // 2, pair, 0)

    pltpu.sync_copy(out_v, out_hbm.at[pl.ds(wid * BPW, BPW)])


def _format_body(x_ref, o_ref):
    x = x_ref[...]
    o_ref[...] = x.reshape(D, CB // 4, 4).transpose((1, 2, 0)).reshape(128, 128)


def _tc_format(table):
    lin = pl.pallas_call(
        _format_body,
        grid=(GRID,),
        in_specs=[pl.BlockSpec((D, CB), lambda g: (0, g))],
        out_specs=pl.BlockSpec((CB * D // 128, 128), lambda g: (g, 0)),
        out_shape=jax.ShapeDtypeStruct((GRID * CB * D // 128, 128),
                                       jnp.float32),
    )(table.T)
    return lin.reshape(VP, D)


@jax.jit
def _sc_call(tok_flat, table):
    mesh = plsc.VectorSubcoreMesh(core_axis_name="c", subcore_axis_name="s")
    return pl.kernel(
        _sc_body,
        out_type=jax.ShapeDtypeStruct((B, D), jnp.float32),
        mesh=mesh,
        compiler_params=pltpu.CompilerParams(
            needs_layout_passes=False, use_tc_tiling_on_sc=False),
        scratch_types=[
            pltpu.VMEM((BPW * S,), jnp.int32),
            pltpu.VMEM((GS, D), jnp.float32),
            pltpu.VMEM((GS, D), jnp.float32),
            pltpu.VMEM((BPW, D), jnp.float32),
            pltpu.SemaphoreType.DMA,
            pltpu.SemaphoreType.DMA,
        ],
    )(tok_flat, table)


def kernel(token_ids, table):
    tok_flat = token_ids.reshape(NW, BPW * S)
    return _sc_call(tok_flat, _tc_format(table))


# speedup vs baseline: 7.3907x; 7.3907x over previous
import functools

import jax
import jax.numpy as jnp
from jax import lax
from jax.experimental import pallas as pl
from jax.experimental.pallas import tpu as pltpu
from jax.experimental.pallas import tpu_sc as plsc

NC = 2
NS = 16
NW = NC * NS
L = 16

B = 4096
S = 200
D = 32
V = 1000000
CB = 4096
GRID = (V + CB - 1) // CB
VP = GRID * CB
BPW = B // NW
G = 4
GS = G * S
NG = BPW // G
STREAMS = ((0, 128), (128, 128), (256, 128), (384, 128),
           (512, 128), (640, 128), (768, 32))


def _sc_body(tok_hbm, table_hbm, out_hbm, tokf_v, rows0, rows1, out_v,
             sem0, sem1):
    cid = lax.axis_index("c")
    sid = lax.axis_index("s")
    wid = sid * NC + cid

    pltpu.sync_copy(tok_hbm.at[wid], tokf_v)

    rows = (rows0, rows1)
    sems = (sem0, sem1)

    def start_gather(g, b):
        tb = g * GS
        for off, ln in STREAMS:
            pltpu.async_copy(table_hbm.at[tokf_v.at[pl.ds(tb + off, ln)]],
                             rows[b].at[pl.ds(off, ln)], sems[b])

    def wait_gather(b):
        pltpu.make_async_copy(table_hbm.at[pl.ds(0, GS)], rows[b], sems[b]).wait()

    zero = jnp.zeros((L,), jnp.float32)
    lane = lax.iota(jnp.int32, 16)
    one = jnp.ones((L,), jnp.float32)

    def compute(g, b):
        rb = rows[b]

        def body2(r, accs):
            accs = list(accs)
            rr = r * 2
            for u in range(G):
                ub = u * S
                for k in range(2):
                    a = u * 4 + k * 2
                    accs[a] = accs[a] + rb[ub + rr + k, pl.ds(0, L)]
                    accs[a + 1] = accs[a + 1] + rb[ub + rr + k, pl.ds(L, L)]
            return tuple(accs)

        accs = lax.fori_loop(0, S // 2, body2, (zero,) * (4 * G))

        for u in range(G):
            s = g * G + u
            o0 = accs[u * 4] + accs[u * 4 + 2]
            o1 = accs[u * 4 + 1] + accs[u * 4 + 3]

            base = s * S
            cnt = zero
            for k in range(S // L):
                t = tokf_v[pl.ds(base + k * L, L)]
                cnt = cnt + jnp.where(t != 0, 1.0, 0.0)
            t = tokf_v[pl.ds(base + S - L, L)]
            cnt = cnt + jnp.where((t != 0) & (lane >= L - S % L), 1.0, 0.0)

            inv = 1.0 / jnp.maximum(jnp.broadcast_to(jnp.sum(cnt), (L,)), one)
            out_v[s, pl.ds(0, L)] = o0 * inv
            out_v[s, pl.ds(L, L)] = o1 * inv

    start_gather(0, 0)
    start_gather(1, 1)

    def pair(i, _):
        g0 = i * 2
        wait_gather(0)
        compute(g0, 0)

        @pl.when(g0 + 2 < NG)
        def _():
            start_gather(g0 + 2, 0)

        wait_gather(1)
        compute(g0 + 1, 1)

        @pl.when(g0 + 3 < NG)
        def _():
            start_gather(g0 + 3, 1)

        return 0

    lax.fori_loop(0, NG // 2, pair, 0)

    pltpu.sync_copy(out_v, out_hbm.at[pl.ds(wid * BPW, BPW)])


def _format_body(x_ref, o_ref):
    o_ref[:, 0:D] = x_ref[...].T


def _tc_format(table):
    lin = pl.pallas_call(
        _format_body,
        grid=(GRID,),
        in_specs=[pl.BlockSpec((D, CB), lambda g: (0, g))],
        out_specs=pl.BlockSpec((CB, 4 * D), lambda g: (g, 0)),
        out_shape=jax.ShapeDtypeStruct((VP, 4 * D), jnp.float32),
    )(table.T)
    return lin.reshape(4 * VP, D)


@jax.jit
def _sc_call(tok_flat, table):
    mesh = plsc.VectorSubcoreMesh(core_axis_name="c", subcore_axis_name="s")
    return pl.kernel(
        _sc_body,
        out_type=jax.ShapeDtypeStruct((B, D), jnp.float32),
        mesh=mesh,
        compiler_params=pltpu.CompilerParams(
            needs_layout_passes=False, use_tc_tiling_on_sc=False),
        scratch_types=[
            pltpu.VMEM((BPW * S,), jnp.int32),
            pltpu.VMEM((GS, D), jnp.float32),
            pltpu.VMEM((GS, D), jnp.float32),
            pltpu.VMEM((BPW, D), jnp.float32),
            pltpu.SemaphoreType.DMA,
            pltpu.SemaphoreType.DMA,
        ],
    )(tok_flat, table)


def kernel(token_ids, table):
    tok_flat = (token_ids * 4).reshape(NW, BPW * S)
    return _sc_call(tok_flat, _tc_format(table))


# speedup vs baseline: 12.7493x; 1.7251x over previous
import functools

import jax
import jax.numpy as jnp
from jax import lax
from jax.experimental import pallas as pl
from jax.experimental.pallas import tpu as pltpu
from jax.experimental.pallas import tpu_sc as plsc

NC = 2
NS = 16
NW = NC * NS
L = 16

B = 4096
S = 200
D = 32
V = 1000000
Q = 1 << 18
CBO = 2048
GRID = Q // CBO
QB = Q // CBO
BPW = B // NW
G = 4
GS = G * S
NG = BPW // G
STREAMS = ((0, 128), (128, 128), (256, 128), (384, 128),
           (512, 128), (640, 128), (768, 32))


def _sc_body(tok_hbm, table_hbm, out_hbm, tokf_v, rows0, rows1, out_v,
             sem0, sem1):
    cid = lax.axis_index("c")
    sid = lax.axis_index("s")
    wid = sid * NC + cid

    pltpu.sync_copy(tok_hbm.at[wid], tokf_v)

    rows = (rows0, rows1)
    sems = (sem0, sem1)

    def start_gather(g, b):
        tb = g * GS
        for off, ln in STREAMS:
            pltpu.async_copy(table_hbm.at[tokf_v.at[pl.ds(tb + off, ln)]],
                             rows[b].at[pl.ds(off, ln)], sems[b])

    def wait_gather(b):
        pltpu.make_async_copy(table_hbm.at[pl.ds(0, GS)], rows[b], sems[b]).wait()

    zero = jnp.zeros((L,), jnp.float32)
    lane = lax.iota(jnp.int32, 16)
    one = jnp.ones((L,), jnp.float32)

    def compute(g, b):
        rb = rows[b]

        def body2(r, accs):
            accs = list(accs)
            rr = r * 2
            for u in range(G):
                ub = u * S
                for k in range(2):
                    a = u * 4 + k * 2
                    accs[a] = accs[a] + rb[ub + rr + k, pl.ds(0, L)]
                    accs[a + 1] = accs[a + 1] + rb[ub + rr + k, pl.ds(L, L)]
            return tuple(accs)

        accs = lax.fori_loop(0, S // 2, body2, (zero,) * (4 * G))

        for u in range(G):
            s = g * G + u
            o0 = accs[u * 4] + accs[u * 4 + 2]
            o1 = accs[u * 4 + 1] + accs[u * 4 + 3]

            base = s * S
            cnt = zero
            for k in range(S // L):
                t = tokf_v[pl.ds(base + k * L, L)]
                cnt = cnt + jnp.where(t != 0, 1.0, 0.0)
            t = tokf_v[pl.ds(base + S - L, L)]
            cnt = cnt + jnp.where((t != 0) & (lane >= L - S % L), 1.0, 0.0)

            inv = 1.0 / jnp.maximum(jnp.broadcast_to(jnp.sum(cnt), (L,)), one)
            out_v[s, pl.ds(0, L)] = o0 * inv
            out_v[s, pl.ds(L, L)] = o1 * inv

    start_gather(0, 0)
    start_gather(1, 1)

    def pair(i, _):
        g0 = i * 2
        wait_gather(0)
        compute(g0, 0)

        @pl.when(g0 + 2 < NG)
        def _():
            start_gather(g0 + 2, 0)

        wait_gather(1)
        compute(g0 + 1, 1)

        @pl.when(g0 + 3 < NG)
        def _():
            start_gather(g0 + 3, 1)

        return 0

    lax.fori_loop(0, NG // 2, pair, 0)

    pltpu.sync_copy(out_v, out_hbm.at[pl.ds(wid * BPW, BPW)])


def _format_body(x0, x1, x2, x3, o_ref):
    x = jnp.concatenate([x0[...], x1[...], x2[...], x3[...]], axis=0)
    o_ref[...] = x.T


def _tc_format(table):
    last = V // CBO
    specs = [pl.BlockSpec(
        (D, CBO), lambda g, u=u: (0, jnp.minimum(u * QB + g, last)))
        for u in range(4)]
    lin = pl.pallas_call(
        _format_body,
        grid=(GRID,),
        in_specs=specs,
        out_specs=pl.BlockSpec((CBO, 4 * D), lambda g: (g, 0)),
        out_shape=jax.ShapeDtypeStruct((Q, 4 * D), jnp.float32),
    )(table.T, table.T, table.T, table.T)
    return lin.reshape(4 * Q, D)


@jax.jit
def _sc_call(tok_flat, table):
    mesh = plsc.VectorSubcoreMesh(core_axis_name="c", subcore_axis_name="s")
    return pl.kernel(
        _sc_body,
        out_type=jax.ShapeDtypeStruct((B, D), jnp.float32),
        mesh=mesh,
        compiler_params=pltpu.CompilerParams(
            needs_layout_passes=False, use_tc_tiling_on_sc=False),
        scratch_types=[
            pltpu.VMEM((BPW * S,), jnp.int32),
            pltpu.VMEM((GS, D), jnp.float32),
            pltpu.VMEM((GS, D), jnp.float32),
            pltpu.VMEM((BPW, D), jnp.float32),
            pltpu.SemaphoreType.DMA,
            pltpu.SemaphoreType.DMA,
        ],
    )(tok_flat, table)


def kernel(token_ids, table):
    tok = ((token_ids & (Q - 1)) << 2) | (token_ids >> 18)
    tok_flat = tok.reshape(NW, BPW * S)
    return _sc_call(tok_flat, _tc_format(table))
